# baseline (speedup 1.0000x reference)
import functools

import jax
import jax.numpy as jnp
from jax import lax
from jax.experimental import pallas as pl
from jax.experimental.pallas import tpu as pltpu
from jax.experimental.pallas import tpu_sc as plsc

N_ROWS = 100000
LD = 16
B = 16384
BLK = 128 // LD
NBLK = N_ROWS // BLK

_info = plsc.get_sparse_core_info()
NC = _info.num_cores
NS = _info.num_subcores
NW = NC * NS
B_PER_W = B // NW
CHUNK = 128
NCHUNK = B_PER_W // CHUNK
NGRP = CHUNK // 16


def _body(idx_hbm, q_mu_hbm, q_ls_hbm, eps_hbm, out_hbm,
          idx_v, bidx_v, mu_b, ls_b, eps_v, out_v, sem, gsem):
    wid = lax.axis_index("s") * NC + lax.axis_index("c")
    base = wid * B_PER_W
    pltpu.sync_copy(idx_hbm.at[pl.ds(wid * NCHUNK, NCHUNK)], idx_v)
    eps_cp = pltpu.async_copy(eps_hbm.at[pl.ds(base, B_PER_W)], eps_v, sem)
    for c in range(NCHUNK):
        def shift(g, _):
            sl = pl.ds(g * 16, 16)
            bidx_v[c, sl] = lax.shift_right_logical(idx_v[c, sl], 3)
            return 0
        lax.fori_loop(0, NGRP, shift, 0)

    def fire(c, buf):
        cp0 = pltpu.async_copy(q_mu_hbm.at[bidx_v.at[c]], mu_b.at[buf], gsem)
        cp1 = pltpu.async_copy(q_ls_hbm.at[bidx_v.at[c]], ls_b.at[buf], gsem)
        return cp0, cp1

    lanes = lax.iota(jnp.int32, 16)
    zeros = jnp.zeros((16,), jnp.int32)
    cps = fire(0, 0)
    eps_cp.wait()
    for c in range(NCHUNK):
        nxt = fire(c + 1, (c + 1) % 2) if c + 1 < NCHUNK else None
        cps[0].wait()
        cps[1].wait()
        buf = c % 2

        def group(g, _):
            vg = idx_v[c, pl.ds(g * 16, 16)]
            for lane in range(16):
                vk = vg[lane]
                off = lax.mul(lax.bitwise_and(vk, 7), 16)
                k = g * 16 + lane
                kk = c * CHUNK + k
                mu_r = mu_b[buf, k, pl.ds(off, 16)]
                ls_r = ls_b[buf, k, pl.ds(off, 16)]
                out_v[kk] = mu_r + jnp.exp(ls_r) * eps_v[kk]
            return 0

        lax.fori_loop(0, NGRP, group, 0)
        cps = nxt
    pltpu.sync_copy(out_v, out_hbm.at[pl.ds(base, B_PER_W)])


@jax.jit
def _run(idx2d, qm2, qls2, eps):
    mesh = plsc.VectorSubcoreMesh(core_axis_name="c", subcore_axis_name="s")
    f = functools.partial(
        pl.kernel,
        mesh=mesh,
        out_type=jax.ShapeDtypeStruct((B, LD), jnp.float32),
        scratch_types=[
            pltpu.VMEM((NCHUNK, CHUNK), jnp.int32),
            pltpu.VMEM((NCHUNK, CHUNK), jnp.int32),
            pltpu.VMEM((2, CHUNK, 128), jnp.float32),
            pltpu.VMEM((2, CHUNK, 128), jnp.float32),
            pltpu.VMEM((B_PER_W, LD), jnp.float32),
            pltpu.VMEM((B_PER_W, LD), jnp.float32),
            pltpu.SemaphoreType.DMA,
            pltpu.SemaphoreType.DMA,
        ],
        compiler_params=pltpu.CompilerParams(use_tc_tiling_on_sc=False),
    )(_body)
    return f(idx2d, qm2, qls2, eps)


def kernel(batch_idx, q_mu, q_log_sigma, prior_loc, prior_var, eps):
    del prior_loc, prior_var
    idx2d = batch_idx.astype(jnp.int32).reshape(NW * NCHUNK, CHUNK)
    qm2 = q_mu.reshape(NBLK, 128)
    qls2 = q_log_sigma.reshape(NBLK, 128)
    return _run(idx2d, qm2, qls2, eps)


# speedup vs baseline: 1.0603x; 1.0603x over previous
import functools

import jax
import jax.numpy as jnp
from jax import lax
from jax.experimental import pallas as pl
from jax.experimental.pallas import tpu as pltpu
from jax.experimental.pallas import tpu_sc as plsc

N_ROWS = 100000
LD = 16
B = 16384

_info = plsc.get_sparse_core_info()
NC = _info.num_cores
NS = _info.num_subcores
NW = NC * NS
B_PER_W = B // NW
CHUNK = 128
NCHUNK = B_PER_W // CHUNK


def _body(idx_hbm, q_mu_hbm, q_ls_hbm, eps_hbm, out_hbm,
          idx_v, mu_v, ls_v, eps_v, out_v, sem):
    wid = lax.axis_index("s") * NC + lax.axis_index("c")
    base = wid * B_PER_W
    pltpu.sync_copy(idx_hbm.at[pl.ds(wid * NCHUNK, NCHUNK)], idx_v)
    eps_cp = pltpu.async_copy(eps_hbm.at[pl.ds(base, B_PER_W)], eps_v, sem)
    cps = []
    for j in range(NCHUNK):
        sl = pl.ds(j * CHUNK, CHUNK)
        cps.append(pltpu.async_copy(q_mu_hbm.at[idx_v.at[j]], mu_v.at[sl], sem))
        cps.append(pltpu.async_copy(q_ls_hbm.at[idx_v.at[j]], ls_v.at[sl], sem))
    eps_cp.wait()
    for cp in cps:
        cp.wait()

    def row(i, _):
        out_v[i] = mu_v[i] + jnp.exp(ls_v[i]) * eps_v[i]
        return 0

    lax.fori_loop(0, B_PER_W, row, 0)
    pltpu.sync_copy(out_v, out_hbm.at[pl.ds(base, B_PER_W)])


@jax.jit
def _run(idx2d, q_mu, q_log_sigma, eps):
    mesh = plsc.VectorSubcoreMesh(core_axis_name="c", subcore_axis_name="s")
    f = functools.partial(
        pl.kernel,
        mesh=mesh,
        out_type=jax.ShapeDtypeStruct((B, LD), jnp.float32),
        scratch_types=[
            pltpu.VMEM((NCHUNK, CHUNK), jnp.int32),
            pltpu.VMEM((B_PER_W, LD), jnp.float32),
            pltpu.VMEM((B_PER_W, LD), jnp.float32),
            pltpu.VMEM((B_PER_W, LD), jnp.float32),
            pltpu.VMEM((B_PER_W, LD), jnp.float32),
            pltpu.SemaphoreType.DMA,
        ],
        compiler_params=pltpu.CompilerParams(use_tc_tiling_on_sc=False),
    )(_body)
    return f(idx2d, q_mu, q_log_sigma, eps)


def kernel(batch_idx, q_mu, q_log_sigma, prior_loc, prior_var, eps):
    del prior_loc, prior_var
    idx2d = batch_idx.astype(jnp.int32).reshape(NW * NCHUNK, CHUNK)
    return _run(idx2d, q_mu, q_log_sigma, eps)
